# Initial kernel scaffold; baseline (speedup 1.0000x reference)
import jax
import jax.numpy as jnp
from jax.experimental import pallas as pl


def kernel(pos, vel, edge_index, Wi, bi, Wm1, bm1, gm1, betam1, Wm2, bm2, gm2, betam2, Wu1, bu1, gu1, betau1, Wu2, bu2, gu2, betau2, Wp, bp):
    raise NotImplementedError("write your pallas kernel here")


# speedup vs baseline: 37.5778x; 37.5778x over previous
import jax
import jax.numpy as jnp
from jax import lax
from jax.experimental import pallas as pl
from jax.experimental.pallas import tpu as pltpu
from jax.experimental.pallas import tpu_sc as plsc

N = 100000
E = 3200000
NC = 2
NS = 16
NW = NC * NS
EW = E // NW
CG = 2000
NCHUNK_G = EW // CG
SCW = 80
SROWS = 10
NROWS_S = E // SCW
RW = NROWS_S // NW
NCHUNK_S = RW // SROWS
EPS = 1e-5

NR_E = E // 16
NR_N = N // 16
BLK = 2000
NBLK = NR_E // BLK

def _mesh():
    return plsc.VectorSubcoreMesh(core_axis_name="c", subcore_axis_name="s")


_f32 = jnp.float32
_sc_params = pltpu.CompilerParams(use_tc_tiling_on_sc=False)


def _kron16(w8):
    return jnp.kron(jnp.eye(16, dtype=_f32), w8.astype(_f32))


def _pat8(v8):
    return jnp.tile(jnp.asarray(v8, _f32).reshape(1, 8), (1, 16)).reshape(1, 128)


def _emb8(w, r0, c0):
    z = jnp.zeros((8, 8), _f32)
    return z.at[r0:r0 + w.shape[0], c0:c0 + w.shape[1]].set(w.astype(_f32))


def _k1_body(x_ref, w_ref, b_ref, h_ref):
    h_ref[...] = (
        jnp.dot(x_ref[...], w_ref[...], preferred_element_type=_f32) + b_ref[...]
    )


def _k1(x8, WiK, bip):
    return pl.pallas_call(
        _k1_body,
        out_shape=jax.ShapeDtypeStruct((NR_N, 128), _f32),
    )(x8, WiK, bip)


def _k2_body(h_hbm, dst_hbm, src_hbm, hd_hbm, hs_hbm, h_sh, idx_v, rows_v, sem):
    c = lax.axis_index("c")
    s = lax.axis_index("s")
    w = s * NC + c

    @pl.when(s == 0)
    def _():
        pltpu.sync_copy(h_hbm, h_sh)

    plsc.subcore_barrier()

    def chunk(i, carry):
        base = pl.multiple_of(w * EW + i * CG, 8)
        pltpu.sync_copy(dst_hbm.at[pl.ds(base, CG)], idx_v)
        pltpu.async_copy(h_sh.at[idx_v], rows_v, sem).wait()
        pltpu.sync_copy(rows_v, hd_hbm.at[pl.ds(base, CG)])
        pltpu.sync_copy(src_hbm.at[pl.ds(base, CG)], idx_v)
        pltpu.async_copy(h_sh.at[idx_v], rows_v, sem).wait()
        pltpu.sync_copy(rows_v, hs_hbm.at[pl.ds(base, CG)])
        return carry

    lax.fori_loop(0, NCHUNK_G, chunk, 0)


def _k2(h8, dst, src):
    fn = pl.kernel(
        _k2_body,
        out_type=[
            jax.ShapeDtypeStruct((E, 8), _f32),
            jax.ShapeDtypeStruct((E, 8), _f32),
        ],
        mesh=_mesh(),
        scratch_types=[
            pltpu.VMEM_SHARED((N, 8), _f32),
            pltpu.VMEM((CG,), jnp.int32),
            pltpu.VMEM((CG, 8), _f32),
            pltpu.SemaphoreType.DMA,
        ],
        compiler_params=_sc_params,
    )
    return fn(h8, dst, src)


def _k3_body(hd_ref, hs_ref, a_ref, b_ref, bm1_ref, st_ref):
    i = pl.program_id(0)

    @pl.when(i == 0)
    def _():
        st_ref[...] = jnp.zeros_like(st_ref)

    hd = hd_ref[...]
    hs = hs_ref[...]
    for g in range(2):
        y = (
            jnp.dot(hd, a_ref[g], preferred_element_type=_f32)
            + jnp.dot(hs, b_ref[g], preferred_element_type=_f32)
            + bm1_ref[0, g]
        )
        st_ref[g, 0:1, :] += jnp.sum(y, axis=0, keepdims=True)
        st_ref[g, 1:2, :] += jnp.sum(y * y, axis=0, keepdims=True)


def _k3(hd8v, hs8v, A8k, B8k, bm1p):
    return pl.pallas_call(
        _k3_body,
        grid=(NBLK,),
        in_specs=[
            pl.BlockSpec((BLK, 128), lambda i: (i, 0)),
            pl.BlockSpec((BLK, 128), lambda i: (i, 0)),
            pl.BlockSpec((2, 128, 128), lambda i: (0, 0, 0)),
            pl.BlockSpec((2, 128, 128), lambda i: (0, 0, 0)),
            pl.BlockSpec((1, 2, 128), lambda i: (0, 0, 0)),
        ],
        out_specs=pl.BlockSpec((2, 2, 128), lambda i: (0, 0, 0)),
        out_shape=jax.ShapeDtypeStruct((2, 2, 128), _f32),
    )(hd8v, hs8v, A8k, B8k, bm1p)


def _k4_body(hd_ref, hs_ref, a_ref, b_ref, bm1_ref, st1_ref, g1_ref, be1_ref,
             c_ref, bm2_ref, t_ref, q_ref, pk_ref,
             x2_ref, mpk_ref, st2_ref):
    i = pl.program_id(0)

    @pl.when(i == 0)
    def _():
        st2_ref[...] = jnp.zeros_like(st2_ref)

    hd = hd_ref[...]
    hs = hs_ref[...]
    T = t_ref[...]
    x2 = jnp.zeros((BLK, 128), _f32) + bm2_ref[...]
    for g in range(2):
        y = (
            jnp.dot(hd, a_ref[g], preferred_element_type=_f32)
            + jnp.dot(hs, b_ref[g], preferred_element_type=_f32)
            + bm1_ref[0, g]
        )
        mean = jnp.dot(st1_ref[g, 0:1, :], T, preferred_element_type=_f32) * (1.0 / E)
        msq = jnp.dot(st1_ref[g, 1:2, :], T, preferred_element_type=_f32) * (1.0 / E)
        var = msq - mean * mean
        a1 = g1_ref[0, g] * lax.rsqrt(var + EPS)
        b1 = jnp.tanh(y * a1 + (be1_ref[0, g] - mean * a1))
        x2 = x2 + jnp.dot(b1, c_ref[g], preferred_element_type=_f32)
    st2_ref[0:1, :] += jnp.sum(x2, axis=0, keepdims=True)
    st2_ref[1:2, :] += jnp.sum(x2 * x2, axis=0, keepdims=True)
    x2_ref[...] = x2
    eq = jnp.where(hd == hs, 1.0, 0.0)
    s8 = jnp.dot(eq, q_ref[...], preferred_element_type=_f32)
    maskf = jnp.where(s8 > 7.5, 0.0, 1.0)
    mpk_ref[...] = jnp.dot(maskf, pk_ref[...], preferred_element_type=_f32)


def _k4(hd8v, hs8v, A8k, B8k, bm1p, st1, g1p, be1p, C8k, bm2p, T8, Q8, PK8):
    return pl.pallas_call(
        _k4_body,
        grid=(NBLK,),
        in_specs=[
            pl.BlockSpec((BLK, 128), lambda i: (i, 0)),
            pl.BlockSpec((BLK, 128), lambda i: (i, 0)),
            pl.BlockSpec((2, 128, 128), lambda i: (0, 0, 0)),
            pl.BlockSpec((2, 128, 128), lambda i: (0, 0, 0)),
            pl.BlockSpec((1, 2, 128), lambda i: (0, 0, 0)),
            pl.BlockSpec((2, 2, 128), lambda i: (0, 0, 0)),
            pl.BlockSpec((1, 2, 128), lambda i: (0, 0, 0)),
            pl.BlockSpec((1, 2, 128), lambda i: (0, 0, 0)),
            pl.BlockSpec((2, 128, 128), lambda i: (0, 0, 0)),
            pl.BlockSpec((1, 128), lambda i: (0, 0)),
            pl.BlockSpec((128, 128), lambda i: (0, 0)),
            pl.BlockSpec((128, 128), lambda i: (0, 0)),
            pl.BlockSpec((128, 16), lambda i: (0, 0)),
        ],
        out_specs=[
            pl.BlockSpec((BLK, 128), lambda i: (i, 0)),
            pl.BlockSpec((BLK, 16), lambda i: (i, 0)),
            pl.BlockSpec((2, 128), lambda i: (0, 0)),
        ],
        out_shape=[
            jax.ShapeDtypeStruct((NR_E, 128), _f32),
            jax.ShapeDtypeStruct((NR_E, 16), _f32),
            jax.ShapeDtypeStruct((2, 128), _f32),
        ],
    )(hd8v, hs8v, A8k, B8k, bm1p, st1, g1p, be1p, C8k, bm2p, T8, Q8, PK8)


def _k5_body(x2_ref, mpk_ref, st2_ref, g2_ref, be2_ref, t_ref, upk_ref,
             sel_ref, one_ref, msg_ref):
    T = t_ref[...]
    mean = jnp.dot(st2_ref[0:1, :], T, preferred_element_type=_f32) * (1.0 / E)
    msq = jnp.dot(st2_ref[1:2, :], T, preferred_element_type=_f32) * (1.0 / E)
    var = msq - mean * mean
    a2 = g2_ref[...] * lax.rsqrt(var + EPS)
    b2 = be2_ref[...] - mean * a2
    t = jnp.tanh(x2_ref[...] * a2 + b2)
    mexp = jnp.dot(mpk_ref[...], upk_ref[...], preferred_element_type=_f32)
    msg_ref[...] = t * mexp * sel_ref[...] + one_ref[...]


def _k5(x2, mpk, st2, g2p, be2p, T8, UPK8, sel03, onep):
    return pl.pallas_call(
        _k5_body,
        grid=(NBLK,),
        in_specs=[
            pl.BlockSpec((BLK, 128), lambda i: (i, 0)),
            pl.BlockSpec((BLK, 16), lambda i: (i, 0)),
            pl.BlockSpec((2, 128), lambda i: (0, 0)),
            pl.BlockSpec((1, 128), lambda i: (0, 0)),
            pl.BlockSpec((1, 128), lambda i: (0, 0)),
            pl.BlockSpec((128, 128), lambda i: (0, 0)),
            pl.BlockSpec((16, 128), lambda i: (0, 0)),
            pl.BlockSpec((1, 128), lambda i: (0, 0)),
            pl.BlockSpec((1, 128), lambda i: (0, 0)),
        ],
        out_specs=pl.BlockSpec((BLK, 128), lambda i: (i, 0)),
        out_shape=jax.ShapeDtypeStruct((NR_E, 128), _f32),
    )(x2, mpk, st2, g2p, be2p, T8, UPK8, sel03, onep)


def _k6_body(msg_hbm, dst_hbm, z8_hbm, acc_out, acc_sh, idx_v, msg_v):
    c = lax.axis_index("c")
    s = lax.axis_index("s")
    w = s * NC + c

    @pl.when(s == 0)
    def _():
        pltpu.sync_copy(z8_hbm, acc_sh)

    plsc.subcore_barrier()

    def chunk(i, carry):
        rbase = pl.multiple_of(w * RW + i * SROWS, 2)
        pltpu.sync_copy(dst_hbm.at[pl.ds(rbase, SROWS)], idx_v)
        pltpu.sync_copy(msg_hbm.at[pl.ds(rbase, SROWS)], msg_v)
        for j in range(SROWS):
            pltpu.sync_copy(msg_v.at[j], acc_sh.at[idx_v.at[j]], add=True)
        return carry

    lax.fori_loop(0, NCHUNK_S, chunk, 0)

    plsc.subcore_barrier()

    @pl.when(s == 0)
    def _():
        pltpu.sync_copy(acc_sh, acc_out.at[c])


def _k6(msg8, dst):
    msg3 = msg8.reshape(NROWS_S, SCW, 8)
    dst2 = dst.reshape(NROWS_S, SCW)
    z8 = jnp.zeros((N, 8), _f32)
    fn = pl.kernel(
        _k6_body,
        out_type=jax.ShapeDtypeStruct((NC, N, 8), _f32),
        mesh=_mesh(),
        scratch_types=[
            pltpu.VMEM_SHARED((N, 8), _f32),
            pltpu.VMEM((SROWS, SCW), jnp.int32),
            pltpu.VMEM((SROWS, SCW, 8), _f32),
        ],
        compiler_params=_sc_params,
    )
    return fn(msg3, dst2, z8)


def _k7_body(h_ref, p_ref, selr_ref, selm_ref, cb_ref, sh_ref, u_ref, bu1_ref,
             g1_ref, be1_ref, c2_ref, bu2_ref, g2_ref, be2_ref, wp_ref, bp_ref,
             t_ref, out_ref):
    T = t_ref[...]
    acc = p_ref[0] + p_ref[1]
    cntb = jnp.dot(acc, cb_ref[...], preferred_element_type=_f32)
    inv = 1.0 / jnp.maximum(cntb, 1.0)
    aggr = acc * (selr_ref[...] + selm_ref[...] * (inv - 1.0))
    u8 = h_ref[...] * selr_ref[...] + jnp.dot(
        aggr, sh_ref[...], preferred_element_type=_f32)
    u2 = jnp.zeros((NR_N, 128), _f32) + bu2_ref[...]
    for g in range(2):
        u1 = (
            jnp.dot(u8, u_ref[g], preferred_element_type=_f32) + bu1_ref[0, g]
        )
        mean = jnp.dot(jnp.sum(u1, 0, keepdims=True), T,
                       preferred_element_type=_f32) * (1.0 / N)
        msq = jnp.dot(jnp.sum(u1 * u1, 0, keepdims=True), T,
                      preferred_element_type=_f32) * (1.0 / N)
        var = msq - mean * mean
        a1 = g1_ref[0, g] * lax.rsqrt(var + EPS)
        b1 = jnp.tanh(u1 * a1 + (be1_ref[0, g] - mean * a1))
        u2 = u2 + jnp.dot(b1, c2_ref[g], preferred_element_type=_f32)
    mean = jnp.dot(jnp.sum(u2, 0, keepdims=True), T,
                   preferred_element_type=_f32) * (1.0 / N)
    msq = jnp.dot(jnp.sum(u2 * u2, 0, keepdims=True), T,
                  preferred_element_type=_f32) * (1.0 / N)
    var = msq - mean * mean
    a2 = g2_ref[...] * lax.rsqrt(var + EPS)
    u2n = jnp.tanh(u2 * a2 + (be2_ref[...] - mean * a2))
    out_ref[...] = (
        jnp.dot(u2n, wp_ref[...], preferred_element_type=_f32) + bp_ref[...]
    )


def _k7(h8, P, selr, selm, CB8, SH4, U8k, bu1p, gu1p, beu1p, C28k, bu2p,
        gu2p, beu2p, Wp8, bp8, T8):
    return pl.pallas_call(
        _k7_body,
        out_shape=jax.ShapeDtypeStruct((NR_N, 128), _f32),
    )(h8, P, selr, selm, CB8, SH4, U8k, bu1p, gu1p, beu1p, C28k, bu2p,
      gu2p, beu2p, Wp8, bp8, T8)


@jax.jit
def kernel(pos, vel, edge_index, Wi, bi, Wm1, bm1, gm1, betam1, Wm2, bm2,
           gm2, betam2, Wu1, bu1, gu1, betau1, Wu2, bu2, gu2, betau2, Wp, bp):
    src = edge_index[0]
    dst = edge_index[1]
    z4 = jnp.zeros((4,), _f32)

    T8 = jnp.kron(jnp.ones((16, 16), _f32), jnp.eye(8, dtype=_f32))
    Q8 = _kron16(jnp.ones((8, 8), _f32))
    PK8 = jnp.kron(jnp.eye(16, dtype=_f32),
                   jnp.eye(8, 1, dtype=_f32))
    UPK8 = _kron16(jnp.ones((1, 8), _f32))
    sel03 = _pat8(jnp.array([1, 1, 1, 1, 0, 0, 0, 0], _f32))
    onep = _pat8(jnp.array([0, 0, 0, 0, 1, 0, 0, 0], _f32))

    x8 = jnp.concatenate([pos, vel, jnp.zeros((N, 4), _f32)], axis=1)
    WiK = _kron16(_emb8(Wi, 0, 0))
    bip = _pat8(jnp.concatenate([bi, jnp.array([1, 0, 0, 0], _f32)]))
    h8 = _k1(x8.reshape(NR_N, 128), WiK, bip)

    hd, hs = _k2(h8.reshape(N, 8), dst, src)
    hd8v = hd.reshape(NR_E, 128)
    hs8v = hs.reshape(NR_E, 128)

    A8k = jnp.stack([_kron16(_emb8(Wm1[0:4, 8 * g:8 * g + 8], 0, 0))
                     for g in range(2)])
    B8k = jnp.stack([_kron16(_emb8(Wm1[4:8, 8 * g:8 * g + 8], 0, 0))
                     for g in range(2)])
    C8k = jnp.stack([_kron16(_emb8(Wm2[8 * g:8 * g + 8, :], 0, 0))
                     for g in range(2)])
    bm1p = jnp.stack([_pat8(bm1[8 * g:8 * g + 8]) for g in range(2)], 1)
    g1p = jnp.stack([_pat8(gm1[8 * g:8 * g + 8]) for g in range(2)], 1)
    be1p = jnp.stack([_pat8(betam1[8 * g:8 * g + 8]) for g in range(2)], 1)
    bm2p = _pat8(jnp.concatenate([bm2, z4]))

    st1 = _k3(hd8v, hs8v, A8k, B8k, bm1p)
    x2, mpk, st2 = _k4(hd8v, hs8v, A8k, B8k, bm1p, st1, g1p, be1p, C8k,
                       bm2p, T8, Q8, PK8)
    msg8 = _k5(x2, mpk, st2, _pat8(jnp.concatenate([gm2, z4])),
               _pat8(jnp.concatenate([betam2, z4])), T8, UPK8, sel03, onep)

    P = _k6(msg8.reshape(E, 8), dst)

    selm = _pat8(jnp.array([0, 0, 1, 1, 0, 0, 0, 0], _f32))
    cb8 = jnp.zeros((8, 8), _f32).at[4, :].set(1.0)
    CB8 = _kron16(cb8)
    sh4 = jnp.zeros((8, 8), _f32).at[0:4, 4:8].set(jnp.eye(4, dtype=_f32))
    SH4 = _kron16(sh4)
    U8k = jnp.stack([_kron16(Wu1[:, 8 * g:8 * g + 8]) for g in range(2)])
    bu1p = jnp.stack([_pat8(bu1[8 * g:8 * g + 8]) for g in range(2)], 1)
    gu1p = jnp.stack([_pat8(gu1[8 * g:8 * g + 8]) for g in range(2)], 1)
    beu1p = jnp.stack([_pat8(betau1[8 * g:8 * g + 8]) for g in range(2)], 1)
    C28k = jnp.stack([_kron16(_emb8(Wu2[8 * g:8 * g + 8, :], 0, 0))
                      for g in range(2)])
    Wp8 = _kron16(_emb8(Wp, 0, 0))
    bp8 = _pat8(jnp.concatenate([bp, jnp.zeros((6,), _f32)]))

    out8 = _k7(h8, P.reshape(NC, NR_N, 128), sel03, selm, CB8, SH4, U8k,
               bu1p, gu1p, beu1p, C28k,
               _pat8(jnp.concatenate([bu2, z4])),
               _pat8(jnp.concatenate([gu2, z4])),
               _pat8(jnp.concatenate([betau2, z4])), Wp8, bp8, T8)
    return out8.reshape(N, 8)[:, :2]


# speedup vs baseline: 39.7028x; 1.0565x over previous
import jax
import jax.numpy as jnp
from jax import lax
from jax.experimental import pallas as pl
from jax.experimental.pallas import tpu as pltpu
from jax.experimental.pallas import tpu_sc as plsc

N = 100000
E = 3200000
NC = 2
NS = 16
NW = NC * NS
EW = E // NW
CG = 2000
NCHUNK_G = EW // CG
SCW = 80
SROWS = 10
NROWS_S = E // SCW
RW = NROWS_S // NW
NCHUNK_S = RW // SROWS
EPS = 1e-5

NR_E = E // 16
NR_N = N // 16
BLK = 2000
NBLK = NR_E // BLK

def _mesh():
    return plsc.VectorSubcoreMesh(core_axis_name="c", subcore_axis_name="s")


_f32 = jnp.float32
_sc_params = pltpu.CompilerParams(use_tc_tiling_on_sc=False)


def _kron16(w8):
    return jnp.kron(jnp.eye(16, dtype=_f32), w8.astype(_f32))


def _pat8(v8):
    return jnp.tile(jnp.asarray(v8, _f32).reshape(1, 8), (1, 16)).reshape(1, 128)


def _emb8(w, r0, c0):
    z = jnp.zeros((8, 8), _f32)
    return z.at[r0:r0 + w.shape[0], c0:c0 + w.shape[1]].set(w.astype(_f32))


def _k1_body(x_ref, w_ref, b_ref, h_ref):
    h_ref[...] = (
        jnp.dot(x_ref[...], w_ref[...], preferred_element_type=_f32) + b_ref[...]
    )


def _k1(x8, WiK, bip):
    return pl.pallas_call(
        _k1_body,
        out_shape=jax.ShapeDtypeStruct((NR_N, 128), _f32),
    )(x8, WiK, bip)


def _k2_body(h_hbm, dst_hbm, src_hbm, hd_hbm, hs_hbm, h_sh,
             idxd0, idxd1, idxs0, idxs1, rd0, rd1, rs0, rs1,
             gsem, wd0, wd1, ws0, ws1):
    c = lax.axis_index("c")
    s = lax.axis_index("s")
    w = s * NC + c
    idxd = (idxd0, idxd1)
    idxs = (idxs0, idxs1)
    rd = (rd0, rd1)
    rs = (rs0, rs1)
    wd = (wd0, wd1)
    ws = (ws0, ws1)

    @pl.when(s == 0)
    def _():
        pltpu.sync_copy(h_hbm, h_sh)

    plsc.subcore_barrier()

    def chunk(i2, carry):
        for b in range(2):
            i = i2 * 2 + b
            base = pl.multiple_of(w * EW + i * CG, 8)

            @pl.when(i2 > 0)
            def _():
                pltpu.make_async_copy(rd[b], hd_hbm.at[pl.ds(base, CG)], wd[b]).wait()
                pltpu.make_async_copy(rs[b], hs_hbm.at[pl.ds(base, CG)], ws[b]).wait()

            pltpu.sync_copy(dst_hbm.at[pl.ds(base, CG)], idxd[b])
            pltpu.async_copy(h_sh.at[idxd[b]], rd[b], gsem).wait()
            pltpu.async_copy(rd[b], hd_hbm.at[pl.ds(base, CG)], wd[b])
            pltpu.sync_copy(src_hbm.at[pl.ds(base, CG)], idxs[b])
            pltpu.async_copy(h_sh.at[idxs[b]], rs[b], gsem).wait()
            pltpu.async_copy(rs[b], hs_hbm.at[pl.ds(base, CG)], ws[b])
        return carry

    lax.fori_loop(0, NCHUNK_G // 2, chunk, 0)

    base0 = pl.multiple_of(w * EW, 8)
    for b in range(2):
        pltpu.make_async_copy(rd[b], hd_hbm.at[pl.ds(base0, CG)], wd[b]).wait()
        pltpu.make_async_copy(rs[b], hs_hbm.at[pl.ds(base0, CG)], ws[b]).wait()


def _k2(h8, dst, src):
    fn = pl.kernel(
        _k2_body,
        out_type=[
            jax.ShapeDtypeStruct((E, 8), _f32),
            jax.ShapeDtypeStruct((E, 8), _f32),
        ],
        mesh=_mesh(),
        scratch_types=[
            pltpu.VMEM_SHARED((N, 8), _f32),
            pltpu.VMEM((CG,), jnp.int32),
            pltpu.VMEM((CG,), jnp.int32),
            pltpu.VMEM((CG,), jnp.int32),
            pltpu.VMEM((CG,), jnp.int32),
            pltpu.VMEM((CG, 8), _f32),
            pltpu.VMEM((CG, 8), _f32),
            pltpu.VMEM((CG, 8), _f32),
            pltpu.VMEM((CG, 8), _f32),
            pltpu.SemaphoreType.DMA,
            pltpu.SemaphoreType.DMA,
            pltpu.SemaphoreType.DMA,
            pltpu.SemaphoreType.DMA,
            pltpu.SemaphoreType.DMA,
        ],
        compiler_params=_sc_params,
    )
    return fn(h8, dst, src)


def _k3_body(hd_ref, hs_ref, a_ref, b_ref, bm1_ref, st_ref):
    i = pl.program_id(0)

    @pl.when(i == 0)
    def _():
        st_ref[...] = jnp.zeros_like(st_ref)

    hd = hd_ref[...]
    hs = hs_ref[...]
    for g in range(2):
        y = (
            jnp.dot(hd, a_ref[g], preferred_element_type=_f32)
            + jnp.dot(hs, b_ref[g], preferred_element_type=_f32)
            + bm1_ref[0, g]
        )
        st_ref[g, 0:1, :] += jnp.sum(y, axis=0, keepdims=True)
        st_ref[g, 1:2, :] += jnp.sum(y * y, axis=0, keepdims=True)


def _k3(hd8v, hs8v, A8k, B8k, bm1p):
    return pl.pallas_call(
        _k3_body,
        grid=(NBLK,),
        in_specs=[
            pl.BlockSpec((BLK, 128), lambda i: (i, 0)),
            pl.BlockSpec((BLK, 128), lambda i: (i, 0)),
            pl.BlockSpec((2, 128, 128), lambda i: (0, 0, 0)),
            pl.BlockSpec((2, 128, 128), lambda i: (0, 0, 0)),
            pl.BlockSpec((1, 2, 128), lambda i: (0, 0, 0)),
        ],
        out_specs=pl.BlockSpec((2, 2, 128), lambda i: (0, 0, 0)),
        out_shape=jax.ShapeDtypeStruct((2, 2, 128), _f32),
    )(hd8v, hs8v, A8k, B8k, bm1p)


def _k4_body(hd_ref, hs_ref, a_ref, b_ref, bm1_ref, st1_ref, g1_ref, be1_ref,
             c_ref, bm2_ref, t_ref, q_ref, pk_ref,
             x2_ref, mpk_ref, st2_ref):
    i = pl.program_id(0)

    @pl.when(i == 0)
    def _():
        st2_ref[...] = jnp.zeros_like(st2_ref)

    hd = hd_ref[...]
    hs = hs_ref[...]
    T = t_ref[...]
    x2 = jnp.zeros((BLK, 128), _f32) + bm2_ref[...]
    for g in range(2):
        y = (
            jnp.dot(hd, a_ref[g], preferred_element_type=_f32)
            + jnp.dot(hs, b_ref[g], preferred_element_type=_f32)
            + bm1_ref[0, g]
        )
        mean = jnp.dot(st1_ref[g, 0:1, :], T, preferred_element_type=_f32) * (1.0 / E)
        msq = jnp.dot(st1_ref[g, 1:2, :], T, preferred_element_type=_f32) * (1.0 / E)
        var = msq - mean * mean
        a1 = g1_ref[0, g] * lax.rsqrt(var + EPS)
        b1 = jnp.tanh(y * a1 + (be1_ref[0, g] - mean * a1))
        x2 = x2 + jnp.dot(b1, c_ref[g], preferred_element_type=_f32)
    st2_ref[0:1, :] += jnp.sum(x2, axis=0, keepdims=True)
    st2_ref[1:2, :] += jnp.sum(x2 * x2, axis=0, keepdims=True)
    x2_ref[...] = x2
    eq = jnp.where(hd == hs, 1.0, 0.0)
    s8 = jnp.dot(eq, q_ref[...], preferred_element_type=_f32)
    maskf = jnp.where(s8 > 7.5, 0.0, 1.0)
    mpk_ref[...] = jnp.dot(maskf, pk_ref[...], preferred_element_type=_f32)


def _k4(hd8v, hs8v, A8k, B8k, bm1p, st1, g1p, be1p, C8k, bm2p, T8, Q8, PK8):
    return pl.pallas_call(
        _k4_body,
        grid=(NBLK,),
        in_specs=[
            pl.BlockSpec((BLK, 128), lambda i: (i, 0)),
            pl.BlockSpec((BLK, 128), lambda i: (i, 0)),
            pl.BlockSpec((2, 128, 128), lambda i: (0, 0, 0)),
            pl.BlockSpec((2, 128, 128), lambda i: (0, 0, 0)),
            pl.BlockSpec((1, 2, 128), lambda i: (0, 0, 0)),
            pl.BlockSpec((2, 2, 128), lambda i: (0, 0, 0)),
            pl.BlockSpec((1, 2, 128), lambda i: (0, 0, 0)),
            pl.BlockSpec((1, 2, 128), lambda i: (0, 0, 0)),
            pl.BlockSpec((2, 128, 128), lambda i: (0, 0, 0)),
            pl.BlockSpec((1, 128), lambda i: (0, 0)),
            pl.BlockSpec((128, 128), lambda i: (0, 0)),
            pl.BlockSpec((128, 128), lambda i: (0, 0)),
            pl.BlockSpec((128, 16), lambda i: (0, 0)),
        ],
        out_specs=[
            pl.BlockSpec((BLK, 128), lambda i: (i, 0)),
            pl.BlockSpec((BLK, 16), lambda i: (i, 0)),
            pl.BlockSpec((2, 128), lambda i: (0, 0)),
        ],
        out_shape=[
            jax.ShapeDtypeStruct((NR_E, 128), _f32),
            jax.ShapeDtypeStruct((NR_E, 16), _f32),
            jax.ShapeDtypeStruct((2, 128), _f32),
        ],
    )(hd8v, hs8v, A8k, B8k, bm1p, st1, g1p, be1p, C8k, bm2p, T8, Q8, PK8)


def _k5_body(x2_ref, mpk_ref, st2_ref, g2_ref, be2_ref, t_ref, upk_ref,
             sel_ref, one_ref, msg_ref):
    T = t_ref[...]
    mean = jnp.dot(st2_ref[0:1, :], T, preferred_element_type=_f32) * (1.0 / E)
    msq = jnp.dot(st2_ref[1:2, :], T, preferred_element_type=_f32) * (1.0 / E)
    var = msq - mean * mean
    a2 = g2_ref[...] * lax.rsqrt(var + EPS)
    b2 = be2_ref[...] - mean * a2
    t = jnp.tanh(x2_ref[...] * a2 + b2)
    mexp = jnp.dot(mpk_ref[...], upk_ref[...], preferred_element_type=_f32)
    msg_ref[...] = t * mexp * sel_ref[...] + one_ref[...]


def _k5(x2, mpk, st2, g2p, be2p, T8, UPK8, sel03, onep):
    return pl.pallas_call(
        _k5_body,
        grid=(NBLK,),
        in_specs=[
            pl.BlockSpec((BLK, 128), lambda i: (i, 0)),
            pl.BlockSpec((BLK, 16), lambda i: (i, 0)),
            pl.BlockSpec((2, 128), lambda i: (0, 0)),
            pl.BlockSpec((1, 128), lambda i: (0, 0)),
            pl.BlockSpec((1, 128), lambda i: (0, 0)),
            pl.BlockSpec((128, 128), lambda i: (0, 0)),
            pl.BlockSpec((16, 128), lambda i: (0, 0)),
            pl.BlockSpec((1, 128), lambda i: (0, 0)),
            pl.BlockSpec((1, 128), lambda i: (0, 0)),
        ],
        out_specs=pl.BlockSpec((BLK, 128), lambda i: (i, 0)),
        out_shape=jax.ShapeDtypeStruct((NR_E, 128), _f32),
    )(x2, mpk, st2, g2p, be2p, T8, UPK8, sel03, onep)


def _k6_body(msg_hbm, dst_hbm, z8_hbm, acc_out, acc_sh, idx_v, msg_v):
    c = lax.axis_index("c")
    s = lax.axis_index("s")
    w = s * NC + c

    @pl.when(s == 0)
    def _():
        pltpu.sync_copy(z8_hbm, acc_sh)

    plsc.subcore_barrier()

    def chunk(i, carry):
        rbase = pl.multiple_of(w * RW + i * SROWS, 2)
        pltpu.sync_copy(dst_hbm.at[pl.ds(rbase, SROWS)], idx_v)
        pltpu.sync_copy(msg_hbm.at[pl.ds(rbase, SROWS)], msg_v)
        for j in range(SROWS):
            pltpu.sync_copy(msg_v.at[j], acc_sh.at[idx_v.at[j]], add=True)
        return carry

    lax.fori_loop(0, NCHUNK_S, chunk, 0)

    plsc.subcore_barrier()

    @pl.when(s == 0)
    def _():
        pltpu.sync_copy(acc_sh, acc_out.at[c])


def _k6(msg8, dst):
    msg3 = msg8.reshape(NROWS_S, SCW, 8)
    dst2 = dst.reshape(NROWS_S, SCW)
    z8 = jnp.zeros((N, 8), _f32)
    fn = pl.kernel(
        _k6_body,
        out_type=jax.ShapeDtypeStruct((NC, N, 8), _f32),
        mesh=_mesh(),
        scratch_types=[
            pltpu.VMEM_SHARED((N, 8), _f32),
            pltpu.VMEM((SROWS, SCW), jnp.int32),
            pltpu.VMEM((SROWS, SCW, 8), _f32),
        ],
        compiler_params=_sc_params,
    )
    return fn(msg3, dst2, z8)


def _k7_body(h_ref, p_ref, selr_ref, selm_ref, cb_ref, sh_ref, u_ref, bu1_ref,
             g1_ref, be1_ref, c2_ref, bu2_ref, g2_ref, be2_ref, wp_ref, bp_ref,
             t_ref, out_ref):
    T = t_ref[...]
    acc = p_ref[0] + p_ref[1]
    cntb = jnp.dot(acc, cb_ref[...], preferred_element_type=_f32)
    inv = 1.0 / jnp.maximum(cntb, 1.0)
    aggr = acc * (selr_ref[...] + selm_ref[...] * (inv - 1.0))
    u8 = h_ref[...] * selr_ref[...] + jnp.dot(
        aggr, sh_ref[...], preferred_element_type=_f32)
    u2 = jnp.zeros((NR_N, 128), _f32) + bu2_ref[...]
    for g in range(2):
        u1 = (
            jnp.dot(u8, u_ref[g], preferred_element_type=_f32) + bu1_ref[0, g]
        )
        mean = jnp.dot(jnp.sum(u1, 0, keepdims=True), T,
                       preferred_element_type=_f32) * (1.0 / N)
        msq = jnp.dot(jnp.sum(u1 * u1, 0, keepdims=True), T,
                      preferred_element_type=_f32) * (1.0 / N)
        var = msq - mean * mean
        a1 = g1_ref[0, g] * lax.rsqrt(var + EPS)
        b1 = jnp.tanh(u1 * a1 + (be1_ref[0, g] - mean * a1))
        u2 = u2 + jnp.dot(b1, c2_ref[g], preferred_element_type=_f32)
    mean = jnp.dot(jnp.sum(u2, 0, keepdims=True), T,
                   preferred_element_type=_f32) * (1.0 / N)
    msq = jnp.dot(jnp.sum(u2 * u2, 0, keepdims=True), T,
                  preferred_element_type=_f32) * (1.0 / N)
    var = msq - mean * mean
    a2 = g2_ref[...] * lax.rsqrt(var + EPS)
    u2n = jnp.tanh(u2 * a2 + (be2_ref[...] - mean * a2))
    out_ref[...] = (
        jnp.dot(u2n, wp_ref[...], preferred_element_type=_f32) + bp_ref[...]
    )


def _k7(h8, P, selr, selm, CB8, SH4, U8k, bu1p, gu1p, beu1p, C28k, bu2p,
        gu2p, beu2p, Wp8, bp8, T8):
    return pl.pallas_call(
        _k7_body,
        out_shape=jax.ShapeDtypeStruct((NR_N, 128), _f32),
    )(h8, P, selr, selm, CB8, SH4, U8k, bu1p, gu1p, beu1p, C28k, bu2p,
      gu2p, beu2p, Wp8, bp8, T8)


@jax.jit
def kernel(pos, vel, edge_index, Wi, bi, Wm1, bm1, gm1, betam1, Wm2, bm2,
           gm2, betam2, Wu1, bu1, gu1, betau1, Wu2, bu2, gu2, betau2, Wp, bp):
    src = edge_index[0]
    dst = edge_index[1]
    z4 = jnp.zeros((4,), _f32)

    T8 = jnp.kron(jnp.ones((16, 16), _f32), jnp.eye(8, dtype=_f32))
    Q8 = _kron16(jnp.ones((8, 8), _f32))
    PK8 = jnp.kron(jnp.eye(16, dtype=_f32),
                   jnp.eye(8, 1, dtype=_f32))
    UPK8 = _kron16(jnp.ones((1, 8), _f32))
    sel03 = _pat8(jnp.array([1, 1, 1, 1, 0, 0, 0, 0], _f32))
    onep = _pat8(jnp.array([0, 0, 0, 0, 1, 0, 0, 0], _f32))

    x8 = jnp.concatenate([pos, vel, jnp.zeros((N, 4), _f32)], axis=1)
    WiK = _kron16(_emb8(Wi, 0, 0))
    bip = _pat8(jnp.concatenate([bi, jnp.array([1, 0, 0, 0], _f32)]))
    h8 = _k1(x8.reshape(NR_N, 128), WiK, bip)

    hd, hs = _k2(h8.reshape(N, 8), dst, src)
    hd8v = hd.reshape(NR_E, 128)
    hs8v = hs.reshape(NR_E, 128)

    A8k = jnp.stack([_kron16(_emb8(Wm1[0:4, 8 * g:8 * g + 8], 0, 0))
                     for g in range(2)])
    B8k = jnp.stack([_kron16(_emb8(Wm1[4:8, 8 * g:8 * g + 8], 0, 0))
                     for g in range(2)])
    C8k = jnp.stack([_kron16(_emb8(Wm2[8 * g:8 * g + 8, :], 0, 0))
                     for g in range(2)])
    bm1p = jnp.stack([_pat8(bm1[8 * g:8 * g + 8]) for g in range(2)], 1)
    g1p = jnp.stack([_pat8(gm1[8 * g:8 * g + 8]) for g in range(2)], 1)
    be1p = jnp.stack([_pat8(betam1[8 * g:8 * g + 8]) for g in range(2)], 1)
    bm2p = _pat8(jnp.concatenate([bm2, z4]))

    st1 = _k3(hd8v, hs8v, A8k, B8k, bm1p)
    x2, mpk, st2 = _k4(hd8v, hs8v, A8k, B8k, bm1p, st1, g1p, be1p, C8k,
                       bm2p, T8, Q8, PK8)
    msg8 = _k5(x2, mpk, st2, _pat8(jnp.concatenate([gm2, z4])),
               _pat8(jnp.concatenate([betam2, z4])), T8, UPK8, sel03, onep)

    P = _k6(msg8.reshape(E, 8), dst)

    selm = _pat8(jnp.array([0, 0, 1, 1, 0, 0, 0, 0], _f32))
    cb8 = jnp.zeros((8, 8), _f32).at[4, :].set(1.0)
    CB8 = _kron16(cb8)
    sh4 = jnp.zeros((8, 8), _f32).at[0:4, 4:8].set(jnp.eye(4, dtype=_f32))
    SH4 = _kron16(sh4)
    U8k = jnp.stack([_kron16(Wu1[:, 8 * g:8 * g + 8]) for g in range(2)])
    bu1p = jnp.stack([_pat8(bu1[8 * g:8 * g + 8]) for g in range(2)], 1)
    gu1p = jnp.stack([_pat8(gu1[8 * g:8 * g + 8]) for g in range(2)], 1)
    beu1p = jnp.stack([_pat8(betau1[8 * g:8 * g + 8]) for g in range(2)], 1)
    C28k = jnp.stack([_kron16(_emb8(Wu2[8 * g:8 * g + 8, :], 0, 0))
                      for g in range(2)])
    Wp8 = _kron16(_emb8(Wp, 0, 0))
    bp8 = _pat8(jnp.concatenate([bp, jnp.zeros((6,), _f32)]))

    out8 = _k7(h8, P.reshape(NC, NR_N, 128), sel03, selm, CB8, SH4, U8k,
               bu1p, gu1p, beu1p, C28k,
               _pat8(jnp.concatenate([bu2, z4])),
               _pat8(jnp.concatenate([gu2, z4])),
               _pat8(jnp.concatenate([betau2, z4])), Wp8, bp8, T8)
    return out8.reshape(N, 8)[:, :2]
